# baseline (speedup 1.0000x reference)
import functools

import jax
import jax.numpy as jnp
from jax import lax
from jax.experimental import pallas as pl
from jax.experimental.pallas import tpu as pltpu
from jax.experimental.pallas import tpu_sc as plsc

EMBED_DIM = 64
SEQ = 200
VCH = 4
CH = VCH * SEQ


def _emb_lookup(idx_flat, table, NV):
    info = plsc.get_sparse_core_info()
    NC, NS = info.num_cores, info.num_subcores
    NW = NC * NS
    v_per_w = NV // NW
    b_per_w = v_per_w * SEQ
    nch = v_per_w // VCH

    mesh = plsc.VectorSubcoreMesh(core_axis_name="c", subcore_axis_name="s")

    @functools.partial(
        pl.kernel,
        mesh=mesh,
        out_type=jax.ShapeDtypeStruct((NV, SEQ, EMBED_DIM), jnp.float32),
        scratch_types=[
            pltpu.VMEM((CH,), jnp.int32),
            pltpu.VMEM((CH,), jnp.int32),
            pltpu.VMEM((CH, EMBED_DIM), jnp.float32),
            pltpu.VMEM((CH, EMBED_DIM), jnp.float32),
            pltpu.SemaphoreType.DMA,
            pltpu.SemaphoreType.DMA,
            pltpu.SemaphoreType.DMA,
            pltpu.SemaphoreType.DMA,
            pltpu.SemaphoreType.DMA,
            pltpu.SemaphoreType.DMA,
        ],
        compiler_params=pltpu.CompilerParams(use_tc_tiling_on_sc=False),
    )
    def emb_kernel(idx_hbm, table_hbm, out_hbm,
                   idx0, idx1, rows0, rows1,
                   si0, si1, sg0, sg1, ss0, ss1):
        idx_v = (idx0, idx1)
        rows_v = (rows0, rows1)
        si = (si0, si1)
        sg = (sg0, sg1)
        ss = (ss0, ss1)
        wid = lax.axis_index("s") * NC + lax.axis_index("c")
        base = wid * b_per_w
        vbase = wid * v_per_w

        def start_idx(j, b):
            pltpu.async_copy(idx_hbm.at[pl.ds(base + j * CH, CH)],
                             idx_v[b], si[b])

        def wait_idx(b):
            pltpu.make_async_copy(idx_hbm.at[pl.ds(base, CH)],
                                  idx_v[b], si[b]).wait()

        def start_gather(b):
            pltpu.async_copy(table_hbm.at[idx_v[b]], rows_v[b], sg[b])

        def wait_gather(b):
            pltpu.make_async_copy(table_hbm.at[idx_v[b]],
                                  rows_v[b], sg[b]).wait()

        def start_store(j, b):
            v0 = vbase + j * VCH
            for k in range(VCH):
                pltpu.async_copy(rows_v[b].at[pl.ds(k * SEQ, SEQ)],
                                 out_hbm.at[v0 + k], ss[b])

        def wait_store(b):
            for k in range(VCH):
                pltpu.make_async_copy(rows_v[b].at[pl.ds(k * SEQ, SEQ)],
                                      out_hbm.at[vbase], ss[b]).wait()

        def step(j, b):
            o = 1 - b
            wait_idx(b)
            wait_store(b)
            start_gather(b)
            wait_gather(o)
            start_idx(j + 1, o)
            start_store(j - 1, o)

        start_idx(0, 0)
        start_idx(1, 1)
        wait_idx(0)
        start_gather(0)
        wait_idx(1)
        start_gather(1)
        wait_gather(0)
        start_idx(2, 0)
        start_store(0, 0)
        step(2, 0)
        def pair(g, carry):
            j = 3 + 2 * g
            step(j, 1)
            step(j + 1, 0)
            return carry
        lax.fori_loop(0, (nch - 4) // 2, pair, 0)
        wait_idx(1)
        wait_store(1)
        start_gather(1)
        wait_gather(0)
        start_store(nch - 2, 0)
        wait_gather(1)
        start_store(nch - 1, 1)
        wait_store(0)
        wait_store(1)

    return emb_kernel(idx_flat, table)


def kernel(visit_node_type, table):
    NV, S = visit_node_type.shape
    idx = visit_node_type.reshape(NV * S).astype(jnp.int32)
    return _emb_lookup(idx, table, NV)
